# baseline (speedup 1.0000x reference)
import functools

import jax
import jax.numpy as jnp
from jax import lax
from jax.experimental import pallas as pl
from jax.experimental.pallas import tpu as pltpu
from jax.experimental.pallas import tpu_sc as plsc


def kernel(indices, table):
    B = indices.shape[0]
    V, D = table.shape
    info = plsc.get_sparse_core_info()
    NC, NS = info.num_cores, info.num_subcores
    NW = NC * NS
    d_per_w = D // NW
    CHUNK = 4096
    n_chunks = B // CHUNK

    tableT = jnp.transpose(table)

    mesh = plsc.VectorSubcoreMesh(core_axis_name="c", subcore_axis_name="s")

    @functools.partial(
        pl.kernel,
        mesh=mesh,
        compiler_params=pltpu.CompilerParams(needs_layout_passes=False),
        out_type=jax.ShapeDtypeStruct((D, B), jnp.float32),
        scratch_types=[
            pltpu.VMEM((B,), jnp.int32),
            pltpu.VMEM((V,), jnp.float32),
            pltpu.VMEM((2, CHUNK), jnp.float32),
            pltpu.SemaphoreType.DMA,
            pltpu.SemaphoreType.DMA,
            pltpu.SemaphoreType.DMA,
        ],
    )
    def gather_kernel(
        idx_hbm, tab_hbm, out_hbm, idx_v, col_v, out_v, isem, csem, osem
    ):
        wid = lax.axis_index("s") * NC + lax.axis_index("c")
        idx_cp = pltpu.async_copy(idx_hbm, idx_v, isem)
        col_cp = pltpu.async_copy(tab_hbm.at[wid * d_per_w], col_v, csem)
        idx_cp.wait()
        col_cp.wait()
        for f in range(d_per_w):
            d = wid * d_per_w + f
            for k in range(n_chunks):
                slot = k % 2
                if f > 0 or k >= 2:
                    pltpu.make_async_copy(
                        out_v.at[slot], out_hbm.at[d, pl.ds(0, CHUNK)], osem
                    ).wait()

                @plsc.parallel_loop(0, CHUNK // 16, unroll=16)
                def body(i):
                    idx16 = idx_v[pl.ds(k * CHUNK + i * 16, 16)]
                    out_v[slot, pl.ds(i * 16, 16)] = plsc.load_gather(
                        col_v, [idx16]
                    )

                pltpu.async_copy(
                    out_v.at[slot], out_hbm.at[d, pl.ds(k * CHUNK, CHUNK)], osem
                )
            if f + 1 < d_per_w:
                pltpu.sync_copy(tab_hbm.at[d + 1], col_v)
        for slot in range(2):
            pltpu.make_async_copy(
                out_v.at[slot],
                out_hbm.at[D - 1, pl.ds(0, CHUNK)],
                osem,
            ).wait()

    outT = gather_kernel(indices, tableT)
    return jnp.transpose(outT)
